# baseline (speedup 1.0000x reference)
import functools

import jax
import jax.numpy as jnp
from jax import lax
from jax.experimental import pallas as pl
from jax.experimental.pallas import tpu as pltpu
from jax.experimental.pallas import tpu_sc as plsc


def _make_gather(batch: int, seq: int, vocab: int, dim: int):
    num_idx = batch * seq
    info = plsc.get_sparse_core_info()
    nw = info.num_cores * info.num_subcores
    b_per_w = num_idx // nw
    ch = min(128, b_per_w)
    n_ch = b_per_w // ch
    wpr = seq // b_per_w

    mesh = plsc.VectorSubcoreMesh(core_axis_name="c", subcore_axis_name="s")

    @functools.partial(
        pl.kernel,
        mesh=mesh,
        out_type=jax.ShapeDtypeStruct((num_idx, dim), jnp.float32),
        scratch_types=[
            pltpu.VMEM((n_ch, ch), jnp.int32),
            pltpu.VMEM((b_per_w, dim), jnp.float32),
            pltpu.SemaphoreType.DMA((n_ch + 1,)),
        ],
    )
    def gather_k(table_hbm, x_hbm, out_hbm, idx_v, rows_v, sems):
        gsem = sems
        wsem = sems.at[n_ch]
        wid = lax.axis_index("s") * info.num_cores + lax.axis_index("c")
        row = wid // wpr
        col = (wid % wpr) * b_per_w
        base = wid * b_per_w
        gathers = []
        for c in range(n_ch):
            pltpu.sync_copy(x_hbm.at[row, pl.ds(col + c * ch, ch)], idx_v.at[c])
            gathers.append(
                pltpu.async_copy(
                    table_hbm.at[idx_v.at[c]],
                    rows_v.at[pl.ds(c * ch, ch)],
                    gsem.at[c],
                )
            )
        writes = []
        for c in range(n_ch):
            gathers[c].wait()
            writes.append(
                pltpu.async_copy(
                    rows_v.at[pl.ds(c * ch, ch)],
                    out_hbm.at[pl.ds(base + c * ch, ch)],
                    wsem,
                )
            )
        for w in writes:
            w.wait()

    return gather_k


def kernel(x, base_embed):
    batch, seq = x.shape
    vocab, dim = base_embed.shape
    gather_k = _make_gather(batch, seq, vocab, dim)
    out = gather_k(base_embed, x)
    return out.reshape(batch, seq, dim)


# speedup vs baseline: 1.0021x; 1.0021x over previous
import functools

import jax
import jax.numpy as jnp
from jax import lax
from jax.experimental import pallas as pl
from jax.experimental.pallas import tpu as pltpu
from jax.experimental.pallas import tpu_sc as plsc


def _make_gather(batch: int, seq: int, vocab: int, dim: int):
    num_idx = batch * seq
    info = plsc.get_sparse_core_info()
    nw = info.num_cores * info.num_subcores
    b_per_w = num_idx // nw
    ch = min(128, b_per_w)
    n_ch = b_per_w // ch
    wpr = seq // b_per_w

    mesh = plsc.VectorSubcoreMesh(core_axis_name="c", subcore_axis_name="s")

    @functools.partial(
        pl.kernel,
        mesh=mesh,
        out_type=jax.ShapeDtypeStruct((num_idx, dim), jnp.float32),
        scratch_types=[
            pltpu.VMEM((n_ch, ch), jnp.int32),
            pltpu.VMEM((b_per_w, dim), jnp.float32),
            pltpu.SemaphoreType.DMA((n_ch,)),
            pltpu.SemaphoreType.DMA,
        ],
    )
    def gather_k(table_hbm, x_hbm, out_hbm, idx_v, rows_v, gsem, wsem):
        wid = lax.axis_index("s") * info.num_cores + lax.axis_index("c")
        row = wid // wpr
        col = (wid % wpr) * b_per_w
        base = wid * b_per_w
        gathers = []
        for c in range(n_ch):
            pltpu.sync_copy(x_hbm.at[row, pl.ds(col + c * ch, ch)], idx_v.at[c])
            gathers.append(
                pltpu.async_copy(
                    table_hbm.at[idx_v.at[c]],
                    rows_v.at[pl.ds(c * ch, ch)],
                    gsem.at[c],
                )
            )
        writes = []
        for c in range(n_ch):
            gathers[c].wait()
            writes.append(
                pltpu.async_copy(
                    rows_v.at[pl.ds(c * ch, ch)],
                    out_hbm.at[pl.ds(base + c * ch, ch)],
                    wsem,
                )
            )
        for w in writes:
            w.wait()

    return gather_k


def kernel(x, base_embed):
    batch, seq = x.shape
    vocab, dim = base_embed.shape
    gather_k = _make_gather(batch, seq, vocab, dim)
    out = gather_k(base_embed, x)
    return out.reshape(batch, seq, dim)
